# baseline (speedup 1.0000x reference)
import functools

import jax
import jax.numpy as jnp
from jax import lax
from jax.experimental import pallas as pl
from jax.experimental.pallas import tpu as pltpu
from jax.experimental.pallas import tpu_sc as plsc

VOCAB = 1000000
EMBED = 64
BATCH = 16384
HIST = 50

NC = 2
NS = 16
NW = NC * NS
BPW = BATCH // NW

BBLK = 8
NBLK = BPW // BBLK
NBUF = 2

_mesh = plsc.VectorSubcoreMesh(core_axis_name="c", subcore_axis_name="s")


@functools.partial(
    pl.kernel,
    mesh=_mesh,
    out_type=jax.ShapeDtypeStruct((BATCH, 56, 128), jnp.float32),
    compiler_params=pltpu.CompilerParams(
        use_tc_tiling_on_sc=False, needs_layout_passes=False),
    scratch_types=[
        pltpu.VMEM((NBUF, BBLK, HIST), jnp.int32),
        pltpu.VMEM((NBUF, BBLK, HIST, 128), jnp.float32),
        pltpu.SemaphoreType.DMA((NBUF,)),
        pltpu.SemaphoreType.DMA,
        pltpu.SemaphoreType.DMA((NBUF,)),
    ],
)
def _emb_lookup(x_hbm, a_hbm, out_hbm, idx_v, rows_v, isem, gsem, wsem):
    wid = lax.axis_index("s") * NC + lax.axis_index("c")
    row0 = wid * BPW

    def start_idx(i, b):
        pltpu.async_copy(
            x_hbm.at[pl.ds(row0 + i * BBLK, BBLK)], idx_v.at[b], isem.at[b])

    def drain_idx(b):
        pltpu.make_async_copy(
            x_hbm.at[pl.ds(0, BBLK)], idx_v.at[b], isem.at[b]).wait()

    def wsrc(b):
        return rows_v.at[b, slice(None), slice(None), pl.ds(0, EMBED)]

    def drain_write(b):
        pltpu.make_async_copy(
            wsrc(b),
            out_hbm.at[pl.ds(0, BBLK), pl.ds(0, HIST), pl.ds(0, EMBED)],
            wsem.at[b]).wait()

    for b in range(NBUF):
        start_idx(b, b)

    def step(i0, carry):
        for b in range(NBUF):
            i = i0 + b
            drain_idx(b)
            @pl.when(i >= NBUF)
            def _():
                drain_write(b)
            copies = [
                pltpu.async_copy(
                    a_hbm.at[idx_v.at[b].at[k]], rows_v.at[b].at[k], gsem)
                for k in range(BBLK)
            ]
            for c in copies:
                c.wait()
            @pl.when(i + NBUF < NBLK)
            def _():
                start_idx(i + NBUF, b)
            pltpu.async_copy(
                wsrc(b),
                out_hbm.at[pl.ds(row0 + i * BBLK, BBLK), pl.ds(0, HIST),
                           pl.ds(0, EMBED)],
                wsem.at[b])
        return carry

    lax.fori_loop(0, NBLK // NBUF, lambda k, c: step(k * NBUF, c), 0)

    for b in range(NBUF):
        drain_write(b)


def kernel(x, A):
    ap = jnp.pad(A, ((0, 0), (0, 128 - EMBED)))
    out = _emb_lookup(x.astype(jnp.int32), ap)
    return out[:, :HIST, :EMBED]


# speedup vs baseline: 1.1052x; 1.1052x over previous
import functools

import jax
import jax.numpy as jnp
from jax import lax
from jax.experimental import pallas as pl
from jax.experimental.pallas import tpu as pltpu
from jax.experimental.pallas import tpu_sc as plsc

VOCAB = 1000000
EMBED = 64
BATCH = 16384
HIST = 50

NC = 2
NS = 16
NW = NC * NS
BPW = BATCH // NW

BBLK = 8
NBLK = BPW // BBLK
NBUF = 2

_mesh = plsc.VectorSubcoreMesh(core_axis_name="c", subcore_axis_name="s")


@functools.partial(
    pl.kernel,
    mesh=_mesh,
    out_type=jax.ShapeDtypeStruct((BATCH, 56, 128), jnp.float32),
    compiler_params=pltpu.CompilerParams(
        use_tc_tiling_on_sc=False, needs_layout_passes=False),
    scratch_types=[
        pltpu.VMEM((NBUF, BBLK, HIST), jnp.int32),
        pltpu.VMEM((NBUF, BBLK, HIST, EMBED), jnp.float32),
        pltpu.SemaphoreType.DMA((NBUF,)),
        pltpu.SemaphoreType.DMA,
        pltpu.SemaphoreType.DMA((NBUF,)),
    ],
)
def _emb_lookup(x_hbm, a_hbm, out_hbm, idx_v, rows_v, isem, gsem, wsem):
    wid = lax.axis_index("s") * NC + lax.axis_index("c")
    row0 = wid * BPW

    def start_idx(i, b):
        pltpu.async_copy(
            x_hbm.at[pl.ds(row0 + i * BBLK, BBLK)], idx_v.at[b], isem.at[b])

    def drain_idx(b):
        pltpu.make_async_copy(
            x_hbm.at[pl.ds(0, BBLK)], idx_v.at[b], isem.at[b]).wait()

    def drain_write(b):
        pltpu.make_async_copy(
            rows_v.at[b],
            out_hbm.at[pl.ds(0, BBLK), pl.ds(0, HIST), pl.ds(0, EMBED)],
            wsem.at[b]).wait()

    for b in range(NBUF):
        start_idx(b, b)

    def step(i0, carry):
        for b in range(NBUF):
            i = i0 + b
            drain_idx(b)
            @pl.when(i >= NBUF)
            def _():
                drain_write(b)
            copies = [
                pltpu.async_copy(
                    a_hbm.at[idx_v.at[b].at[k]], rows_v.at[b].at[k], gsem)
                for k in range(BBLK)
            ]
            for c in copies:
                c.wait()
            @pl.when(i + NBUF < NBLK)
            def _():
                start_idx(i + NBUF, b)
            pltpu.async_copy(
                rows_v.at[b],
                out_hbm.at[pl.ds(row0 + i * BBLK, BBLK), pl.ds(0, HIST),
                           pl.ds(0, EMBED)],
                wsem.at[b])
        return carry

    lax.fori_loop(0, NBLK // NBUF, lambda k, c: step(k * NBUF, c), 0)

    for b in range(NBUF):
        drain_write(b)


def kernel(x, A):
    out = _emb_lookup(x.astype(jnp.int32), A)
    return out[:, :HIST, :EMBED]
